# baseline (speedup 1.0000x reference)
import functools

import jax
import jax.numpy as jnp
from jax import lax
from jax.experimental import pallas as pl
from jax.experimental.pallas import tpu as pltpu
from jax.experimental.pallas import tpu_sc as plsc

_PAD = 0


def kernel(src, pos_table):
    N, S = src.shape
    _, E = pos_table.shape

    info = plsc.get_sparse_core_info()
    NC, NS, L = info.num_cores, info.num_subcores, info.num_lanes
    NW = NC * NS
    rows_per_w = S // NW
    cols_per_w = (N * S) // NW
    n_vec = cols_per_w // L

    mesh = plsc.VectorSubcoreMesh(core_axis_name="c", subcore_axis_name="s")

    @functools.partial(
        pl.kernel,
        mesh=mesh,
        out_type=(
            jax.ShapeDtypeStruct((N, S, E), pos_table.dtype),
            jax.ShapeDtypeStruct((N, S), jnp.int32),
        ),
        scratch_types=[
            pltpu.VMEM((cols_per_w,), jnp.int32),
            pltpu.VMEM((cols_per_w,), jnp.int32),
        ],
    )
    def sc_k(src_hbm, table_hbm, pos_out, mask_out, sbuf, mbuf):
        wid = lax.axis_index("s") * NC + lax.axis_index("c")

        base = wid * rows_per_w
        for n in range(N):
            pltpu.sync_copy(
                table_hbm.at[pl.ds(base, rows_per_w)],
                pos_out.at[n, pl.ds(base, rows_per_w)],
            )

        row = wid // (S // cols_per_w)
        col = (wid % (S // cols_per_w)) * cols_per_w
        pltpu.sync_copy(src_hbm.at[row, pl.ds(col, cols_per_w)], sbuf)

        def body(i, _):
            v = sbuf[pl.ds(i * L, L)]
            mbuf[pl.ds(i * L, L)] = jnp.where(v == _PAD, 1, 0).astype(jnp.int32)
            return 0

        lax.fori_loop(0, n_vec, body, 0)
        pltpu.sync_copy(mbuf, mask_out.at[row, pl.ds(col, cols_per_w)])

    pos_emb, mask_i32 = sc_k(src, pos_table)
    return pos_emb, mask_i32.astype(jnp.bool_)


# speedup vs baseline: 60.3026x; 60.3026x over previous
import functools

import jax
import jax.numpy as jnp
from jax import lax
from jax.experimental import pallas as pl
from jax.experimental.pallas import tpu as pltpu
from jax.experimental.pallas import tpu_sc as plsc

_PAD = 0
_NCH = 8


def _bcast_body(table_hbm, out_hbm, buf, rsem, wsem):
    N = out_hbm.shape[0]
    S = table_hbm.shape[0]
    ch = S // _NCH
    reads = [
        pltpu.make_async_copy(
            table_hbm.at[pl.ds(c * ch, ch)], buf.at[pl.ds(c * ch, ch)], rsem.at[c]
        )
        for c in range(_NCH)
    ]
    for r in reads:
        r.start()
    writes = []
    for c in range(_NCH):
        reads[c].wait()
        for n in range(N):
            w = pltpu.make_async_copy(
                buf.at[pl.ds(c * ch, ch)],
                out_hbm.at[n, pl.ds(c * ch, ch)],
                wsem.at[c, n],
            )
            w.start()
            writes.append(w)
    for w in writes:
        w.wait()


def _make_sc_mask(N, S):
    info = plsc.get_sparse_core_info()
    NC, NS, L = info.num_cores, info.num_subcores, info.num_lanes
    NW = NC * NS
    cols_per_w = (N * S) // NW
    n_vec = cols_per_w // L
    mesh = plsc.VectorSubcoreMesh(core_axis_name="c", subcore_axis_name="s")

    @functools.partial(
        pl.kernel,
        mesh=mesh,
        out_type=jax.ShapeDtypeStruct((N, S), jnp.int32),
        scratch_types=[
            pltpu.VMEM((cols_per_w,), jnp.int32),
            pltpu.VMEM((cols_per_w,), jnp.int32),
        ],
    )
    def sc_mask(src_hbm, mask_out, sbuf, mbuf):
        wid = lax.axis_index("s") * NC + lax.axis_index("c")
        row = wid // (S // cols_per_w)
        col = (wid % (S // cols_per_w)) * cols_per_w
        pltpu.sync_copy(src_hbm.at[row, pl.ds(col, cols_per_w)], sbuf)

        def body(i, _):
            v = sbuf[pl.ds(i * L, L)]
            mbuf[pl.ds(i * L, L)] = jnp.where(v == _PAD, 1, 0).astype(jnp.int32)
            return 0

        lax.fori_loop(0, n_vec, body, 0)
        pltpu.sync_copy(mbuf, mask_out.at[row, pl.ds(col, cols_per_w)])

    return sc_mask


def kernel(src, pos_table):
    N, S = src.shape
    _, E = pos_table.shape

    pos_emb = pl.pallas_call(
        _bcast_body,
        in_specs=[pl.BlockSpec(memory_space=pl.ANY)],
        out_specs=pl.BlockSpec(memory_space=pl.ANY),
        out_shape=jax.ShapeDtypeStruct((N, S, E), pos_table.dtype),
        scratch_shapes=[
            pltpu.VMEM((S, E), pos_table.dtype),
            pltpu.SemaphoreType.DMA((_NCH,)),
            pltpu.SemaphoreType.DMA((_NCH, N)),
        ],
    )(pos_table)

    mask_i32 = _make_sc_mask(N, S)(src)
    return pos_emb, mask_i32.astype(jnp.bool_)
